# baseline (speedup 1.0000x reference)
import jax
import jax.numpy as jnp
from jax.experimental import pallas as pl

N = 10000
E = 320000
B = 20
FEAT = 128
H = 128


def _ln(x, g, b):
    mu = jnp.mean(x, axis=-1, keepdims=True)
    var = jnp.var(x, axis=-1, keepdims=True)
    return (x - mu) / jnp.sqrt(var + 1e-5) * g + b


def _seg_softmax(v, idx, n):
    m = jax.ops.segment_max(v, idx, num_segments=n)
    m = jnp.where(jnp.isfinite(m), m, 0.0)
    e = jnp.exp(v - m[idx])
    s = jax.ops.segment_sum(e, idx, num_segments=n)
    return e / (s[idx] + 1e-16)


def _genconv(x, src, dst, p, l, n):
    msg = jax.nn.relu(x[src]) + 1e-7
    alpha = _seg_softmax(msg * p['t%d' % l], dst, n)
    agg = jax.ops.segment_sum(msg * alpha, dst, num_segments=n)
    out = agg + x
    h = out @ p['Wm1_%d' % l] + p['bm1_%d' % l]
    h = _ln(h, p['g1_%d' % l], p['be1_%d' % l])
    h = jax.nn.relu(h)
    return h @ p['Wm2_%d' % l] + p['bm2_%d' % l]


def _bias_add_kernel(x_ref, b_ref, o_ref):
    o_ref[...] = x_ref[...] + b_ref[...]


def kernel(x, edge_index, edge_latent, y, params):
    p = params
    src = edge_index[0]
    dst = edge_index[1]
    x = jax.nn.relu(x @ p['Wfc'] + p['bfc'])
    x_ = x
    h = _genconv(x, src, dst, p, 0, N)
    x_ = jnp.concatenate([x_, h], axis=-1)
    x = h
    for l in (1, 2):
        hh = _genconv(x, src, dst, p, l, N)
        hh = _ln(hh, p['gn_%d' % l], p['bn_%d' % l])
        hh = jax.nn.relu(hh)
        x = x + hh
        x_ = jnp.concatenate([x_, x], axis=-1)
    hp = x_.reshape(B, 500, 4 * H)
    hp = jax.nn.relu(hp @ p['Wphi'] + p['bphi'])
    a = jnp.tanh(hp @ p['Wa'] + p['ba'])
    g = jax.nn.sigmoid(hp @ p['Wb'] + p['bb'])
    A = (a * g) @ p['Wc'] + p['bc']
    A = jnp.swapaxes(A, -1, -2)
    A = jax.nn.softmax(A, axis=-1)
    h2 = jnp.matmul(A, hp)
    bag = jax.nn.relu(h2 @ p['Wrho'] + p['brho'])[:, 0, :]
    logits = bag @ p['Wcls']
    out = pl.pallas_call(
        _bias_add_kernel,
        out_shape=jax.ShapeDtypeStruct(logits.shape, logits.dtype),
    )(logits, jnp.broadcast_to(p['bcls'], logits.shape))
    return out


# speedup vs baseline: 5.2929x; 5.2929x over previous
import functools

import jax
import jax.numpy as jnp
from jax import lax
from jax.experimental import pallas as pl
from jax.experimental.pallas import tpu as pltpu
from jax.experimental.pallas import tpu_sc as plsc

N = 10000
E = 320000
B = 20
FEAT = 128
H = 128

NC = 2
NS = 16
NW = NC * NS
CHUNK = 128
EDGES_PER_W = 10112
E_PAD = EDGES_PER_W * NW
N_PAD = 10112
ROWS_PER_S = N_PAD // NS


def _sc_segsum(table0, table1, src_pad, dst_pad, zrows):
    mesh = plsc.VectorSubcoreMesh(core_axis_name="c", subcore_axis_name="s")

    @functools.partial(
        pl.kernel,
        mesh=mesh,
        out_type=[
            jax.ShapeDtypeStruct((NC, N_PAD, 128), jnp.float32),
            jax.ShapeDtypeStruct((NC, N_PAD, 128), jnp.float32),
        ],
        scratch_types=[
            pltpu.VMEM((CHUNK,), jnp.int32),
            pltpu.VMEM((CHUNK,), jnp.int32),
            pltpu.VMEM((CHUNK, 128), jnp.float32),
            pltpu.VMEM_SHARED((N_PAD, 128), jnp.float32),
            pltpu.SemaphoreType.DMA,
        ],
    )
    def k(t0_hbm, t1_hbm, src_hbm, dst_hbm, z_hbm, out0_hbm, out1_hbm,
          sidx, didx, gbuf, acc, sem):
        cid = lax.axis_index("c")
        sid = lax.axis_index("s")
        wid = cid * NS + sid
        ebase = wid * EDGES_PER_W
        rbase = sid * ROWS_PER_S

        for p, (tab, outp) in enumerate(((t0_hbm, out0_hbm), (t1_hbm, out1_hbm))):
            pltpu.sync_copy(z_hbm.at[pl.ds(rbase, ROWS_PER_S)],
                            acc.at[pl.ds(rbase, ROWS_PER_S)])
            plsc.subcore_barrier()

            def body(g, carry):
                off = ebase + g * CHUNK
                pltpu.sync_copy(src_hbm.at[pl.ds(off, CHUNK)], sidx)
                pltpu.sync_copy(dst_hbm.at[pl.ds(off, CHUNK)], didx)
                pltpu.async_copy(tab.at[sidx], gbuf, sem).wait()
                pltpu.sync_copy(gbuf, acc.at[didx], add=True)
                return carry

            lax.fori_loop(0, EDGES_PER_W // CHUNK, body, 0)
            plsc.subcore_barrier()
            pltpu.sync_copy(acc.at[pl.ds(rbase, ROWS_PER_S)],
                            outp.at[cid, pl.ds(rbase, ROWS_PER_S)])
            if p == 0:
                plsc.subcore_barrier()

    return k(table0, table1, src_pad, dst_pad, zrows)


def _ln(x, g, b):
    mu = jnp.mean(x, axis=-1, keepdims=True)
    var = jnp.var(x, axis=-1, keepdims=True)
    return (x - mu) / jnp.sqrt(var + 1e-5) * g + b


def _genconv(x, src_pad, dst_pad, zrows, p, l):
    r = jax.nn.relu(x) + 1e-7
    P = jnp.exp(r * p['t%d' % l])
    Q = r * P
    table0 = jnp.concatenate([P[:, :64], Q[:, :64]], axis=1)
    table1 = jnp.concatenate([P[:, 64:], Q[:, 64:]], axis=1)
    out0, out1 = _sc_segsum(table0, table1, src_pad, dst_pad, zrows)
    o0 = out0[0] + out0[1]
    o1 = out1[0] + out1[1]
    S = jnp.concatenate([o0[:N, :64], o1[:N, :64]], axis=1)
    W = jnp.concatenate([o0[:N, 64:], o1[:N, 64:]], axis=1)
    agg = W / (S + 1e-16)
    out = agg + x
    h = out @ p['Wm1_%d' % l] + p['bm1_%d' % l]
    h = _ln(h, p['g1_%d' % l], p['be1_%d' % l])
    h = jax.nn.relu(h)
    return h @ p['Wm2_%d' % l] + p['bm2_%d' % l]


def kernel(x, edge_index, edge_latent, y, params):
    p = params
    src = edge_index[0].astype(jnp.int32)
    dst = edge_index[1].astype(jnp.int32)
    pad = E_PAD - E
    src_pad = jnp.concatenate([src, jnp.zeros((pad,), jnp.int32)])
    dst_pad = jnp.concatenate([dst, jnp.full((pad,), N, jnp.int32)])
    zrows = jnp.zeros((N_PAD, 128), jnp.float32)

    x = jax.nn.relu(x @ p['Wfc'] + p['bfc'])
    x_ = x
    h = _genconv(x, src_pad, dst_pad, zrows, p, 0)
    x_ = jnp.concatenate([x_, h], axis=-1)
    x = h
    for l in (1, 2):
        hh = _genconv(x, src_pad, dst_pad, zrows, p, l)
        hh = _ln(hh, p['gn_%d' % l], p['bn_%d' % l])
        hh = jax.nn.relu(hh)
        x = x + hh
        x_ = jnp.concatenate([x_, x], axis=-1)
    hp = x_.reshape(B, 500, 4 * H)
    hp = jax.nn.relu(hp @ p['Wphi'] + p['bphi'])
    a = jnp.tanh(hp @ p['Wa'] + p['ba'])
    g = jax.nn.sigmoid(hp @ p['Wb'] + p['bb'])
    A = (a * g) @ p['Wc'] + p['bc']
    A = jnp.swapaxes(A, -1, -2)
    A = jax.nn.softmax(A, axis=-1)
    h2 = jnp.matmul(A, hp)
    bag = jax.nn.relu(h2 @ p['Wrho'] + p['brho'])[:, 0, :]
    return bag @ p['Wcls'] + p['bcls']
